# baseline (speedup 1.0000x reference)
import functools

import jax
import jax.numpy as jnp
from jax import lax
from jax.experimental import pallas as pl
from jax.experimental.pallas import tpu as pltpu
from jax.experimental.pallas import tpu_sc as plsc

B, L, D, V, NB = 4096, 50, 128, 100000, 512
NC, NS = 2, 16
NW = NC * NS
BPW = B // NW
FLATW = BPW * L
CB = 8
CHUNK = CB * L
NCH = BPW // CB
SUBS = ((0, 128), (128, 128), (256, 128), (384, 16))
LANES = 16


def _pool_kernel(idg_hbm, ide_hbm, val_hbm, gt_hbm, et_hbm, out_hbm,
                 idg_lin, ide_lin, val_lin, cnt_v, t0_v, z_v, r0_v,
                 gbuf, sums_v, sem):
    wid = lax.axis_index("s") * NC + lax.axis_index("c")
    base = wid * FLATW

    pltpu.sync_copy(idg_hbm.at[pl.ds(base, FLATW)], idg_lin)
    pltpu.sync_copy(ide_hbm.at[pl.ds(base, FLATW)], ide_lin)
    pltpu.sync_copy(val_hbm.at[pl.ds(base, FLATW)], val_lin)

    def mask_body(k, carry):
        sl = pl.ds(k * LANES, LANES)
        v = val_lin[sl]
        idg_lin[sl] = idg_lin[sl] * v
        ide_lin[sl] = ide_lin[sl] * v
        return carry

    z_v[...] = jnp.zeros((LANES,), jnp.int32)
    pltpu.async_copy(gt_hbm.at[z_v], r0_v, sem).wait()
    for dc in range(D // LANES):
        t0_v[pl.ds(dc * LANES, LANES)] = r0_v[0, pl.ds(dc * LANES, LANES)]
    pltpu.async_copy(et_hbm.at[z_v], r0_v, sem).wait()
    for dc in range(D // LANES):
        sl = pl.ds(dc * LANES, LANES)
        t0_v[sl] = t0_v[sl] + r0_v[0, sl]

    iota16 = lax.iota(jnp.int32, LANES)
    for bc in range(BPW // LANES):
        bvec50 = (iota16 + bc * LANES) * L
        def cnt_body(l, cnt):
            return cnt + plsc.load_gather(val_lin, [bvec50 + l])
        cnt = lax.fori_loop(0, L, cnt_body, jnp.zeros((LANES,), jnp.int32))
        cnt_v[pl.ds(bc * LANES, LANES)] = cnt

    def chunk_body(c, carry):
        cb = c * CHUNK
        for table_hbm, idx_lin, first in ((gt_hbm, idg_lin, True),
                                          (et_hbm, ide_lin, False)):
            cps = [pltpu.async_copy(
                       table_hbm.at[idx_lin.at[pl.ds(cb + off, sz)]],
                       gbuf.at[pl.ds(off, sz)], sem)
                   for off, sz in SUBS]
            for cp in cps:
                cp.wait()
            pass
        return carry
    lax.fori_loop(0, NCH, chunk_body, 0)

    for bc in range(BPW // LANES):
        bvec = iota16 + bc * LANES
        cntf = cnt_v[pl.ds(bc * LANES, LANES)].astype(jnp.float32)
        inv = 1.0 / jnp.maximum(cntf, 1.0)
        spur = jnp.float32(L) - cntf
        def scale_body(d, carry):
            dsplat = jnp.full((LANES,), d, jnp.int32)
            t0d = plsc.load_gather(t0_v, [dsplat])
            s = plsc.load_gather(sums_v, [bvec, dsplat])
            plsc.store_scatter(sums_v, [bvec, dsplat], (s - spur * t0d) * inv)
            return carry
        lax.fori_loop(0, D, scale_body, 0)

    pltpu.sync_copy(sums_v, out_hbm.at[pl.ds(wid * BPW, BPW)])


@jax.jit
def _sc_pool(idg, ide, val, gene_table, expr_table):
    mesh = plsc.VectorSubcoreMesh(core_axis_name="c", subcore_axis_name="s",
                                  num_cores=NC, num_subcores=NS)
    return pl.kernel(
        _pool_kernel,
        out_type=jax.ShapeDtypeStruct((B, D), jnp.float32),
        mesh=mesh,
        scratch_types=[
            pltpu.VMEM((FLATW,), jnp.int32),
            pltpu.VMEM((FLATW,), jnp.int32),
            pltpu.VMEM((FLATW,), jnp.int32),
            pltpu.VMEM((BPW,), jnp.int32),
            pltpu.VMEM((D,), jnp.float32),
            pltpu.VMEM((LANES,), jnp.int32),
            pltpu.VMEM((LANES, D), jnp.float32),
            pltpu.VMEM((512, D), jnp.float32),
            pltpu.VMEM((BPW, D), jnp.float32),
            pltpu.SemaphoreType.DMA,
        ],
        compiler_params=pltpu.CompilerParams(needs_layout_passes=False),
    )(idg, ide, val, gene_table, expr_table)


def kernel(identity_inputs, expression_inputs, attention_mask, gene_table,
           expr_table):
    idg = identity_inputs.astype(jnp.int32).reshape(-1)
    ide = expression_inputs.astype(jnp.int32).reshape(-1)
    val = (~attention_mask).astype(jnp.int32).reshape(-1)
    return _sc_pool(idg, ide, val,
                    gene_table.astype(jnp.float32),
                    expr_table.astype(jnp.float32))


# speedup vs baseline: 1.2775x; 1.2775x over previous
import jax
import jax.numpy as jnp
from jax import lax
from jax.experimental import pallas as pl
from jax.experimental.pallas import tpu as pltpu
from jax.experimental.pallas import tpu_sc as plsc

B, L, D, V, NB = 4096, 50, 128, 100000, 512
NC, NS = 2, 16
NW = NC * NS
BPW = B // NW
FLATW = BPW * L
CB = 4
CHUNK = CB * L
NCH = BPW // CB
SUB2 = ((0, 128), (128, 72))
LANES = 16
DC = D // LANES


def _pool_kernel(idg_hbm, ide_hbm, valf_hbm, gt_hbm, et_hbm, out_hbm,
                 idg_lin, ide_lin, valf_lin, inv_v, buf0, buf1, sums_v,
                 semg, seme):
    wid = lax.axis_index("s") * NC + lax.axis_index("c")
    base = wid * FLATW

    pltpu.sync_copy(idg_hbm.at[pl.ds(base, FLATW)], idg_lin)
    pltpu.sync_copy(ide_hbm.at[pl.ds(base, FLATW)], ide_lin)
    pltpu.sync_copy(valf_hbm.at[pl.ds(base, FLATW)], valf_lin)

    iota16 = lax.iota(jnp.int32, LANES)
    for bc in range(BPW // LANES):
        bvec50 = (iota16 + bc * LANES) * L
        def cnt_body(l, cnt):
            return cnt + plsc.load_gather(valf_lin, [bvec50 + l])
        cnt = lax.fori_loop(0, L, cnt_body, jnp.zeros((LANES,), jnp.float32))
        inv_v[pl.ds(bc * LANES, LANES)] = 1.0 / jnp.maximum(cnt, 1.0)

    def fire(g, b):
        tbl = gt_hbm if b == 0 else et_hbm
        idx = idg_lin if b == 0 else ide_lin
        buf = buf0 if b == 0 else buf1
        sem = semg if b == 0 else seme
        for off, sz in SUB2:
            pltpu.async_copy(tbl.at[idx.at[pl.ds(g * CHUNK + off, sz)]],
                             buf.at[pl.ds(off, sz)], sem)

    fire(0, 0)
    fire(0, 1)

    def job_body(g, carry):
        cbase = g * CHUNK
        for b in (0, 1):
            tbl = gt_hbm if b == 0 else et_hbm
            idx = idg_lin if b == 0 else ide_lin
            buf = buf0 if b == 0 else buf1
            sem = semg if b == 0 else seme
            for off, sz in SUB2:
                pltpu.make_async_copy(
                    tbl.at[idx.at[pl.ds(cbase + off, sz)]],
                    buf.at[pl.ds(off, sz)], sem).wait()
            for r in range(CB):
                def acc_body(l, accs):
                    p = r * L + l
                    vs = plsc.load_gather(
                        valf_lin, [jnp.full((LANES,), cbase + p, jnp.int32)])
                    return tuple(
                        accs[dc] + buf[p, pl.ds(dc * LANES, LANES)] * vs
                        for dc in range(DC))
                accs = lax.fori_loop(
                    0, L, acc_body,
                    tuple(jnp.zeros((LANES,), jnp.float32)
                          for _ in range(DC)))
                rowg = g * CB + r
                if b == 0:
                    for dc in range(DC):
                        sums_v[rowg, pl.ds(dc * LANES, LANES)] = accs[dc]
                else:
                    iv = plsc.load_gather(
                        inv_v, [jnp.full((LANES,), rowg, jnp.int32)])
                    for dc in range(DC):
                        sl = pl.ds(dc * LANES, LANES)
                        sums_v[rowg, sl] = (sums_v[rowg, sl] + accs[dc]) * iv
            @pl.when(g + 1 < NCH)
            def _():
                fire(g + 1, b)
        return carry
    lax.fori_loop(0, NCH, job_body, 0)

    pltpu.sync_copy(sums_v, out_hbm.at[pl.ds(wid * BPW, BPW)])


@jax.jit
def _sc_pool(idg, ide, valf, gene_table, expr_table):
    mesh = plsc.VectorSubcoreMesh(core_axis_name="c", subcore_axis_name="s",
                                  num_cores=NC, num_subcores=NS)
    return pl.kernel(
        _pool_kernel,
        out_type=jax.ShapeDtypeStruct((B, D), jnp.float32),
        mesh=mesh,
        scratch_types=[
            pltpu.VMEM((FLATW,), jnp.int32),
            pltpu.VMEM((FLATW,), jnp.int32),
            pltpu.VMEM((FLATW,), jnp.float32),
            pltpu.VMEM((BPW,), jnp.float32),
            pltpu.VMEM((CHUNK, D), jnp.float32),
            pltpu.VMEM((CHUNK, D), jnp.float32),
            pltpu.VMEM((BPW, D), jnp.float32),
            pltpu.SemaphoreType.DMA,
            pltpu.SemaphoreType.DMA,
        ],
        compiler_params=pltpu.CompilerParams(needs_layout_passes=False),
    )(idg, ide, valf, gene_table, expr_table)


def kernel(identity_inputs, expression_inputs, attention_mask, gene_table,
           expr_table):
    idg = identity_inputs.astype(jnp.int32).reshape(-1)
    ide = expression_inputs.astype(jnp.int32).reshape(-1)
    valf = (~attention_mask).astype(jnp.float32).reshape(-1)
    return _sc_pool(idg, ide, valf,
                    gene_table.astype(jnp.float32),
                    expr_table.astype(jnp.float32))


# speedup vs baseline: 1.9287x; 1.5098x over previous
import jax
import jax.numpy as jnp
from jax import lax
from jax.experimental import pallas as pl
from jax.experimental.pallas import tpu as pltpu
from jax.experimental.pallas import tpu_sc as plsc

B, L, D, V, NB = 4096, 50, 128, 100000, 512
NC, NS = 2, 16
NW = NC * NS
BPW = B // NW
FLATW = BPW * L
CB = 4
CHUNK = CB * L
NCH = BPW // CB
SUB2 = ((0, 128), (128, 72))
LANES = 16
DC = D // LANES
HW = LANES * NB


def _pool_kernel(idg_hbm, ide_hbm, valf_hbm, gt_hbm, gs_out, hist_out,
                 idg_lin, ide_lin, valf_lin, buf0, buf1, sums_v, hbuf,
                 semg, seme, semh):
    wid = lax.axis_index("s") * NC + lax.axis_index("c")
    base = wid * FLATW
    iota16 = lax.iota(jnp.int32, LANES)

    pltpu.sync_copy(idg_hbm.at[pl.ds(base, FLATW)], idg_lin)
    pltpu.sync_copy(ide_hbm.at[pl.ds(base, FLATW)], ide_lin)
    pltpu.sync_copy(valf_hbm.at[pl.ds(base, FLATW)], valf_lin)

    def fire(c, b):
        buf = buf0 if b == 0 else buf1
        sem = semg if b == 0 else seme
        for off, sz in SUB2:
            pltpu.async_copy(gt_hbm.at[idg_lin.at[pl.ds(c * CHUNK + off, sz)]],
                             buf.at[pl.ds(off, sz)], sem)

    fire(0, 0)
    fire(1, 1)

    zero16 = jnp.zeros((LANES,), jnp.float32)
    lanebase = iota16 * NB
    for bc in range(BPW // LANES):
        pb = bc % 2
        hrow = hbuf.at[pl.ds(pb * HW, HW)]
        if bc >= 2:
            hb_prev = (wid * BPW + (bc - 2) * LANES) * NB
            pltpu.make_async_copy(hbuf.at[pl.ds(pb * HW, HW)],
                                  hist_out.at[pl.ds(hb_prev, HW)],
                                  semh).wait()
        def zero_body(k, carry):
            for u in range(8):
                hrow[pl.ds((k * 8 + u) * LANES, LANES)] = zero16
            return carry
        lax.fori_loop(0, HW // (8 * LANES), zero_body, 0)
        bvec50 = (iota16 + bc * LANES) * L
        def fill_body(l, carry):
            bins = plsc.load_gather(ide_lin, [bvec50 + l])
            vals = plsc.load_gather(valf_lin, [bvec50 + l])
            plsc.addupdate_scatter(hrow, [lanebase + bins], vals)
            return carry
        lax.fori_loop(0, L, fill_body, 0)
        hb = (wid * BPW + bc * LANES) * NB
        pltpu.async_copy(hbuf.at[pl.ds(pb * HW, HW)],
                         hist_out.at[pl.ds(hb, HW)], semh)
    for bc in (BPW // LANES - 2, BPW // LANES - 1):
        hb = (wid * BPW + bc * LANES) * NB
        pltpu.make_async_copy(hbuf.at[pl.ds((bc % 2) * HW, HW)],
                              hist_out.at[pl.ds(hb, HW)], semh).wait()

    def pair_body(gg, carry):
        for b in (0, 1):
            c = gg * 2 + b
            cbase = c * CHUNK
            buf = buf0 if b == 0 else buf1
            sem = semg if b == 0 else seme
            for off, sz in SUB2:
                pltpu.make_async_copy(
                    gt_hbm.at[idg_lin.at[pl.ds(cbase + off, sz)]],
                    buf.at[pl.ds(off, sz)], sem).wait()
            for r in range(CB):
                def acc_body(l, accs):
                    p = r * L + l
                    vs = plsc.load_gather(
                        valf_lin, [jnp.full((LANES,), cbase + p, jnp.int32)])
                    return tuple(
                        accs[dc] + buf[p, pl.ds(dc * LANES, LANES)] * vs
                        for dc in range(DC))
                accs = lax.fori_loop(
                    0, L, acc_body,
                    tuple(jnp.zeros((LANES,), jnp.float32)
                          for _ in range(DC)))
                rowg = c * CB + r
                for dc in range(DC):
                    sums_v[rowg, pl.ds(dc * LANES, LANES)] = accs[dc]
            @pl.when(c + 2 < NCH)
            def _():
                fire(c + 2, b)
        return carry
    lax.fori_loop(0, NCH // 2, pair_body, 0)

    pltpu.sync_copy(sums_v, gs_out.at[pl.ds(wid * BPW, BPW)])


@jax.jit
def _sc_gather_hist(idg, ide, valf, gene_table):
    mesh = plsc.VectorSubcoreMesh(core_axis_name="c", subcore_axis_name="s",
                                  num_cores=NC, num_subcores=NS)
    return pl.kernel(
        _pool_kernel,
        out_type=(jax.ShapeDtypeStruct((B, D), jnp.float32),
                  jax.ShapeDtypeStruct((B * NB,), jnp.float32)),
        mesh=mesh,
        scratch_types=[
            pltpu.VMEM((FLATW,), jnp.int32),
            pltpu.VMEM((FLATW,), jnp.int32),
            pltpu.VMEM((FLATW,), jnp.float32),
            pltpu.VMEM((CHUNK, D), jnp.float32),
            pltpu.VMEM((CHUNK, D), jnp.float32),
            pltpu.VMEM((BPW, D), jnp.float32),
            pltpu.VMEM((2 * HW,), jnp.float32),
            pltpu.SemaphoreType.DMA,
            pltpu.SemaphoreType.DMA,
            pltpu.SemaphoreType.DMA,
        ],
        compiler_params=pltpu.CompilerParams(needs_layout_passes=False),
    )(idg, ide, valf, gene_table)


def _combine_body(gs_ref, h_ref, et_ref, o_ref):
    h = h_ref[...]
    cnt = jnp.sum(h, axis=1, keepdims=True)
    acc = lax.dot_general(h, et_ref[...], (((1,), (0,)), ((), ())),
                          preferred_element_type=jnp.float32)
    o_ref[...] = (gs_ref[...] + acc) / jnp.maximum(cnt, 1.0)


RB = 256


@jax.jit
def _tc_combine(gs, hist, expr_table):
    return pl.pallas_call(
        _combine_body,
        out_shape=jax.ShapeDtypeStruct((B, D), jnp.float32),
        grid=(B // RB,),
        in_specs=[
            pl.BlockSpec((RB, D), lambda i: (i, 0)),
            pl.BlockSpec((RB, NB), lambda i: (i, 0)),
            pl.BlockSpec((NB, D), lambda i: (0, 0)),
        ],
        out_specs=pl.BlockSpec((RB, D), lambda i: (i, 0)),
    )(gs, hist, expr_table)


def kernel(identity_inputs, expression_inputs, attention_mask, gene_table,
           expr_table):
    idg = identity_inputs.astype(jnp.int32).reshape(-1)
    ide = expression_inputs.astype(jnp.int32).reshape(-1)
    valf = (~attention_mask).astype(jnp.float32).reshape(-1)
    gs, histf = _sc_gather_hist(idg, ide, valf,
                                gene_table.astype(jnp.float32))
    return _tc_combine(gs, histf.reshape(B, NB),
                       expr_table.astype(jnp.float32))


# speedup vs baseline: 2.4349x; 1.2624x over previous
import jax
import jax.numpy as jnp
from jax import lax
from jax.experimental import pallas as pl
from jax.experimental.pallas import tpu as pltpu
from jax.experimental.pallas import tpu_sc as plsc

B, L, D, V, NB = 4096, 50, 128, 100000, 512
NC, NS = 2, 16
NW = NC * NS
BPW = B // NW
FLATW = BPW * L
BLK = 128
NBLK = FLATW // BLK
LANES = 16
DC = D // LANES
HW = LANES * NB


def _sread(ref, i, iota16):
    g = plsc.load_gather(ref, [jnp.full((LANES,), i, jnp.int32)])
    return jnp.sum(jnp.where(iota16 == 0, g, 0))


def _pool_kernel(idg_hbm, ide_hbm, valf_hbm, gt_hbm, gs_out, hist_out,
                 idg_lin, ide_lin, valf_lin, cidx, scnt, bbuf0, bbuf1,
                 sums_v, hbuf, semg, seme, semh):
    wid = lax.axis_index("s") * NC + lax.axis_index("c")
    base = wid * FLATW
    iota16 = lax.iota(jnp.int32, LANES)

    pltpu.sync_copy(idg_hbm.at[pl.ds(base, FLATW)], idg_lin)
    pltpu.sync_copy(ide_hbm.at[pl.ds(base, FLATW)], ide_lin)
    pltpu.sync_copy(valf_hbm.at[pl.ds(base, FLATW)], valf_lin)

    def compact_body(k, pos):
        sl = pl.ds(k * LANES, LANES)
        cidx[sl] = k * LANES + iota16
        m = valf_lin[sl] != 0.0
        mi = m.astype(jnp.int32)
        excl = plsc.cumsum(mi) - mi
        plsc.store_scatter(cidx, [excl + pos], idg_lin[sl], mask=m)
        return pos + jnp.sum(mi)
    nc = lax.fori_loop(0, FLATW // LANES, compact_body, jnp.int32(0))

    def fire(blk, b):
        buf = bbuf0 if b == 0 else bbuf1
        sem = semg if b == 0 else seme
        @pl.when(blk * BLK < nc)
        def _():
            pltpu.async_copy(gt_hbm.at[cidx.at[pl.ds(blk * BLK, BLK)]],
                             buf, sem)

    def drain(blk, b):
        buf = bbuf0 if b == 0 else bbuf1
        sem = semg if b == 0 else seme
        @pl.when(blk * BLK < nc)
        def _():
            pltpu.make_async_copy(gt_hbm.at[cidx.at[pl.ds(blk * BLK, BLK)]],
                                  buf, sem).wait()

    fire(0, 0)
    fire(1, 1)

    rbase = jnp.int32(0)
    for bc in range(BPW // LANES):
        bvec50 = (iota16 + bc * LANES) * L
        def cnt_body(l, cnt):
            return cnt + plsc.load_gather(valf_lin, [bvec50 + l])
        cntf = lax.fori_loop(0, L, cnt_body, jnp.zeros((LANES,), jnp.float32))
        cnt = cntf.astype(jnp.int32)
        excl = plsc.cumsum(cnt) - cnt
        scnt[pl.ds(bc * LANES, LANES)] = excl + rbase
        rbase = rbase + jnp.sum(cnt)
    scnt[pl.ds(BPW, LANES)] = jnp.full((LANES,), rbase, jnp.int32)

    zero16 = jnp.zeros((LANES,), jnp.float32)
    lanebase = iota16 * D
    for bc in range(BPW // LANES):
        pb = bc % 2
        hrow = hbuf.at[pl.ds(pb * HW, HW)]
        if bc >= 2:
            for j in range(NB // D):
                hb = j * (B * D) + (wid * BPW + (bc - 2) * LANES) * D
                pltpu.make_async_copy(
                    hbuf.at[pl.ds(pb * HW + j * (LANES * D), LANES * D)],
                    hist_out.at[pl.ds(hb, LANES * D)], semh).wait()
        def zero_body(k, carry):
            for u in range(8):
                hrow[pl.ds((k * 8 + u) * LANES, LANES)] = zero16
            return carry
        lax.fori_loop(0, HW // (8 * LANES), zero_body, 0)
        bvec50 = (iota16 + bc * LANES) * L
        def fill_body(l, carry):
            bins = plsc.load_gather(ide_lin, [bvec50 + l])
            vals = plsc.load_gather(valf_lin, [bvec50 + l])
            j = lax.shift_right_logical(bins, 7)
            rem = lax.bitwise_and(bins, jnp.int32(127))
            plsc.addupdate_scatter(hrow, [j * (LANES * D) + lanebase + rem],
                                   vals)
            return carry
        lax.fori_loop(0, L, fill_body, 0)
        for j in range(NB // D):
            hb = j * (B * D) + (wid * BPW + bc * LANES) * D
            pltpu.async_copy(
                hbuf.at[pl.ds(pb * HW + j * (LANES * D), LANES * D)],
                hist_out.at[pl.ds(hb, LANES * D)], semh)
    for bc in (BPW // LANES - 2, BPW // LANES - 1):
        for j in range(NB // D):
            hb = j * (B * D) + (wid * BPW + bc * LANES) * D
            pltpu.make_async_copy(
                hbuf.at[pl.ds((bc % 2) * HW + j * (LANES * D), LANES * D)],
                hist_out.at[pl.ds(hb, LANES * D)], semh).wait()

    def zs_body(r, carry):
        for dc in range(DC):
            sums_v[r, pl.ds(dc * LANES, LANES)] = zero16
        return carry
    lax.fori_loop(0, BPW, zs_body, 0)

    zeros8 = tuple(jnp.zeros((LANES,), jnp.float32) for _ in range(DC))

    def pair_body(fbp, st):
        for b in (0, 1):
            fb = fbp * 2 + b
            buf = bbuf0 if b == 0 else bbuf1
            drain(fb, b)
            blockend = jnp.minimum((fb + 1) * BLK, nc)
            fb0 = fb * BLK

            def wcond(s):
                return s[1] < blockend

            def wbody(s):
                r, p = s
                e_r = _sread(scnt, r + 1, iota16)
                pe = jnp.minimum(e_r, blockend)
                off0 = p - fb0

                def abody(i, accs):
                    return tuple(
                        accs[dc] + buf[off0 + i, pl.ds(dc * LANES, LANES)]
                        for dc in range(DC))
                accs = lax.fori_loop(0, pe - p, abody, zeros8)
                for dc in range(DC):
                    sl = pl.ds(dc * LANES, LANES)
                    sums_v[r, sl] = sums_v[r, sl] + accs[dc]
                return (jnp.where(pe == e_r, r + 1, r), pe)

            st = lax.while_loop(wcond, wbody, st)
            fire(fb + 2, b)
        return st
    lax.fori_loop(0, NBLK // 2, pair_body, (jnp.int32(0), jnp.int32(0)))

    pltpu.sync_copy(sums_v, gs_out.at[pl.ds(wid * BPW, BPW)])


@jax.jit
def _sc_gather_hist(idg, ide, valf, gene_table):
    mesh = plsc.VectorSubcoreMesh(core_axis_name="c", subcore_axis_name="s",
                                  num_cores=NC, num_subcores=NS)
    return pl.kernel(
        _pool_kernel,
        out_type=(jax.ShapeDtypeStruct((B, D), jnp.float32),
                  jax.ShapeDtypeStruct((B * NB,), jnp.float32)),
        mesh=mesh,
        scratch_types=[
            pltpu.VMEM((FLATW,), jnp.int32),
            pltpu.VMEM((FLATW,), jnp.int32),
            pltpu.VMEM((FLATW,), jnp.float32),
            pltpu.VMEM((FLATW,), jnp.int32),
            pltpu.VMEM((BPW + LANES,), jnp.int32),
            pltpu.VMEM((BLK, D), jnp.float32),
            pltpu.VMEM((BLK, D), jnp.float32),
            pltpu.VMEM((BPW, D), jnp.float32),
            pltpu.VMEM((2 * HW,), jnp.float32),
            pltpu.SemaphoreType.DMA,
            pltpu.SemaphoreType.DMA,
            pltpu.SemaphoreType.DMA,
        ],
        compiler_params=pltpu.CompilerParams(needs_layout_passes=False),
    )(idg, ide, valf, gene_table)


RB = 512


def _combine_body(gs_ref, h_ref, et_ref, o_ref):
    h3 = h_ref[...]
    cnt = jnp.sum(h3, axis=(0, 2))[:, None]
    acc = gs_ref[...]
    for j in range(NB // D):
        acc = acc + lax.dot_general(
            h3[j], et_ref[pl.ds(j * D, D), :],
            (((1,), (0,)), ((), ())), preferred_element_type=jnp.float32)
    o_ref[...] = acc / jnp.maximum(cnt, 1.0)


@jax.jit
def _tc_combine(gs, histf, expr_table):
    hist3 = histf.reshape(NB // D, B, D)
    return pl.pallas_call(
        _combine_body,
        out_shape=jax.ShapeDtypeStruct((B, D), jnp.float32),
        grid=(B // RB,),
        in_specs=[
            pl.BlockSpec((RB, D), lambda i: (i, 0)),
            pl.BlockSpec((NB // D, RB, D), lambda i: (0, i, 0)),
            pl.BlockSpec((NB, D), lambda i: (0, 0)),
        ],
        out_specs=pl.BlockSpec((RB, D), lambda i: (i, 0)),
    )(gs, hist3, expr_table)


def kernel(identity_inputs, expression_inputs, attention_mask, gene_table,
           expr_table):
    idg = identity_inputs.astype(jnp.int32).reshape(-1)
    ide = expression_inputs.astype(jnp.int32).reshape(-1)
    valf = (~attention_mask).astype(jnp.float32).reshape(-1)
    gs, histf = _sc_gather_hist(idg, ide, valf,
                                gene_table.astype(jnp.float32))
    return _tc_combine(gs, histf, expr_table.astype(jnp.float32))


# speedup vs baseline: 2.4691x; 1.0141x over previous
import jax
import jax.numpy as jnp
from jax import lax
from jax.experimental import pallas as pl
from jax.experimental.pallas import tpu as pltpu
from jax.experimental.pallas import tpu_sc as plsc

B, L, D, V, NB = 4096, 50, 128, 100000, 512
NC, NS = 2, 16
NW = NC * NS
BPW = B // NW
FLATW = BPW * L
BLK = 128
NBLK = FLATW // BLK
LANES = 16
DC = D // LANES
HW = LANES * NB


def _sread(ref, i, iota16):
    g = plsc.load_gather(ref, [jnp.full((LANES,), i, jnp.int32)])
    return jnp.sum(jnp.where(iota16 == 0, g, 0))


def _pool_kernel(arr_hbm, gt_hbm, gs_out, hist_out,
                 idg_lin, ide_lin, vali_lin, cidx, scnt, bbuf0, bbuf1,
                 sums_v, hbuf, semg, seme, semh):
    wid = lax.axis_index("s") * NC + lax.axis_index("c")
    base = wid * FLATW
    iota16 = lax.iota(jnp.int32, LANES)

    pltpu.sync_copy(arr_hbm.at[pl.ds(base, FLATW)], idg_lin)
    pltpu.sync_copy(arr_hbm.at[pl.ds(B * L + base, FLATW)], ide_lin)
    pltpu.sync_copy(arr_hbm.at[pl.ds(2 * B * L + base, FLATW)], vali_lin)

    def compact_body(k, pos):
        sl = pl.ds(k * LANES, LANES)
        cidx[sl] = k * LANES + iota16
        m = vali_lin[sl] != 0
        mi = m.astype(jnp.int32)
        excl = plsc.cumsum(mi) - mi
        plsc.store_scatter(cidx, [excl + pos], idg_lin[sl], mask=m)
        return pos + jnp.sum(mi)
    nc = lax.fori_loop(0, FLATW // LANES, compact_body, jnp.int32(0))

    def fire(blk, b):
        buf = bbuf0 if b == 0 else bbuf1
        sem = semg if b == 0 else seme
        @pl.when(blk * BLK < nc)
        def _():
            pltpu.async_copy(gt_hbm.at[cidx.at[pl.ds(blk * BLK, BLK)]],
                             buf, sem)

    def drain(blk, b):
        buf = bbuf0 if b == 0 else bbuf1
        sem = semg if b == 0 else seme
        @pl.when(blk * BLK < nc)
        def _():
            pltpu.make_async_copy(gt_hbm.at[cidx.at[pl.ds(blk * BLK, BLK)]],
                                  buf, sem).wait()

    fire(0, 0)
    fire(1, 1)

    rbase = jnp.int32(0)
    for bc in range(BPW // LANES):
        bvec50 = (iota16 + bc * LANES) * L
        def cnt_body(l, cnt):
            return cnt + plsc.load_gather(vali_lin, [bvec50 + l])
        cnt = lax.fori_loop(0, L, cnt_body, jnp.zeros((LANES,), jnp.int32))
        excl = plsc.cumsum(cnt) - cnt
        scnt[pl.ds(bc * LANES, LANES)] = excl + rbase
        rbase = rbase + jnp.sum(cnt)
    scnt[pl.ds(BPW, LANES)] = jnp.full((LANES,), rbase, jnp.int32)

    zero16 = jnp.zeros((LANES,), jnp.float32)
    lanebase = iota16 * D
    for bc in range(BPW // LANES):
        pb = bc % 2
        hrow = hbuf.at[pl.ds(pb * HW, HW)]
        if bc >= 2:
            for j in range(NB // D):
                hb = j * (B * D) + (wid * BPW + (bc - 2) * LANES) * D
                pltpu.make_async_copy(
                    hbuf.at[pl.ds(pb * HW + j * (LANES * D), LANES * D)],
                    hist_out.at[pl.ds(hb, LANES * D)], semh).wait()
        def zero_body(k, carry):
            for u in range(8):
                hrow[pl.ds((k * 8 + u) * LANES, LANES)] = zero16
            return carry
        lax.fori_loop(0, HW // (8 * LANES), zero_body, 0)
        bvec50 = (iota16 + bc * LANES) * L
        def fill_body(l, carry):
            bins = plsc.load_gather(ide_lin, [bvec50 + l])
            vals = plsc.load_gather(vali_lin, [bvec50 + l]).astype(jnp.float32)
            j = lax.shift_right_logical(bins, 7)
            rem = lax.bitwise_and(bins, jnp.int32(127))
            plsc.addupdate_scatter(hrow, [j * (LANES * D) + lanebase + rem],
                                   vals)
            return carry
        lax.fori_loop(0, L, fill_body, 0)
        for j in range(NB // D):
            hb = j * (B * D) + (wid * BPW + bc * LANES) * D
            pltpu.async_copy(
                hbuf.at[pl.ds(pb * HW + j * (LANES * D), LANES * D)],
                hist_out.at[pl.ds(hb, LANES * D)], semh)
    for bc in (BPW // LANES - 2, BPW // LANES - 1):
        for j in range(NB // D):
            hb = j * (B * D) + (wid * BPW + bc * LANES) * D
            pltpu.make_async_copy(
                hbuf.at[pl.ds((bc % 2) * HW + j * (LANES * D), LANES * D)],
                hist_out.at[pl.ds(hb, LANES * D)], semh).wait()

    def zs_body(r, carry):
        for dc in range(DC):
            sums_v[r, pl.ds(dc * LANES, LANES)] = zero16
        return carry
    lax.fori_loop(0, BPW, zs_body, 0)

    zeros8 = tuple(jnp.zeros((LANES,), jnp.float32) for _ in range(DC))

    def pair_body(fbp, st):
        for b in (0, 1):
            fb = fbp * 2 + b
            buf = bbuf0 if b == 0 else bbuf1
            drain(fb, b)
            blockend = jnp.minimum((fb + 1) * BLK, nc)
            fb0 = fb * BLK

            def wcond(s):
                return s[1] < blockend

            def wbody(s):
                r, p = s
                e_r = _sread(scnt, r + 1, iota16)
                pe = jnp.minimum(e_r, blockend)
                off0 = p - fb0

                def abody(i, accs):
                    return tuple(
                        accs[dc] + buf[off0 + i, pl.ds(dc * LANES, LANES)]
                        for dc in range(DC))
                accs = lax.fori_loop(0, pe - p, abody, zeros8)
                for dc in range(DC):
                    sl = pl.ds(dc * LANES, LANES)
                    sums_v[r, sl] = sums_v[r, sl] + accs[dc]
                return (jnp.where(pe == e_r, r + 1, r), pe)

            st = lax.while_loop(wcond, wbody, st)
            fire(fb + 2, b)
        return st
    lax.fori_loop(0, NBLK // 2, pair_body, (jnp.int32(0), jnp.int32(0)))

    pltpu.sync_copy(sums_v, gs_out.at[pl.ds(wid * BPW, BPW)])


@jax.jit
def _sc_gather_hist(arr, gene_table):
    mesh = plsc.VectorSubcoreMesh(core_axis_name="c", subcore_axis_name="s",
                                  num_cores=NC, num_subcores=NS)
    return pl.kernel(
        _pool_kernel,
        out_type=(jax.ShapeDtypeStruct((B, D), jnp.float32),
                  jax.ShapeDtypeStruct((B * NB,), jnp.float32)),
        mesh=mesh,
        scratch_types=[
            pltpu.VMEM((FLATW,), jnp.int32),
            pltpu.VMEM((FLATW,), jnp.int32),
            pltpu.VMEM((FLATW,), jnp.int32),
            pltpu.VMEM((FLATW,), jnp.int32),
            pltpu.VMEM((BPW + LANES,), jnp.int32),
            pltpu.VMEM((BLK, D), jnp.float32),
            pltpu.VMEM((BLK, D), jnp.float32),
            pltpu.VMEM((BPW, D), jnp.float32),
            pltpu.VMEM((2 * HW,), jnp.float32),
            pltpu.SemaphoreType.DMA,
            pltpu.SemaphoreType.DMA,
            pltpu.SemaphoreType.DMA,
        ],
        compiler_params=pltpu.CompilerParams(needs_layout_passes=False),
    )(arr, gene_table)


RB = 512


def _combine_body(gs_ref, h_ref, et_ref, o_ref):
    h3 = h_ref[...]
    cnt = jnp.sum(h3, axis=(0, 2))[:, None]
    acc = gs_ref[...]
    for j in range(NB // D):
        acc = acc + lax.dot_general(
            h3[j], et_ref[pl.ds(j * D, D), :],
            (((1,), (0,)), ((), ())), preferred_element_type=jnp.float32)
    o_ref[...] = acc / jnp.maximum(cnt, 1.0)


@jax.jit
def _tc_combine(gs, histf, expr_table):
    hist3 = histf.reshape(NB // D, B, D)
    return pl.pallas_call(
        _combine_body,
        out_shape=jax.ShapeDtypeStruct((B, D), jnp.float32),
        grid=(B // RB,),
        in_specs=[
            pl.BlockSpec((RB, D), lambda i: (i, 0)),
            pl.BlockSpec((NB // D, RB, D), lambda i: (0, i, 0)),
            pl.BlockSpec((NB, D), lambda i: (0, 0)),
        ],
        out_specs=pl.BlockSpec((RB, D), lambda i: (i, 0)),
    )(gs, hist3, expr_table)


def kernel(identity_inputs, expression_inputs, attention_mask, gene_table,
           expr_table):
    arr = jnp.concatenate([
        identity_inputs.astype(jnp.int32).reshape(-1),
        expression_inputs.astype(jnp.int32).reshape(-1),
        (~attention_mask).astype(jnp.int32).reshape(-1),
    ])
    gs, histf = _sc_gather_hist(arr, gene_table.astype(jnp.float32))
    return _tc_combine(gs, histf, expr_table.astype(jnp.float32))


# speedup vs baseline: 2.7200x; 1.1016x over previous
import jax
import jax.numpy as jnp
from jax import lax
from jax.experimental import pallas as pl
from jax.experimental.pallas import tpu as pltpu
from jax.experimental.pallas import tpu_sc as plsc

B, L, D, V, NB = 4096, 50, 128, 100000, 512
NC, NS = 2, 16
NW = NC * NS
BPW = B // NW
FLATW = BPW * L
BLK = 128
NBLK = FLATW // BLK
LANES = 16
DC = D // LANES
HW = LANES * NB


def _sread(ref, i, iota16):
    g = plsc.load_gather(ref, [jnp.full((LANES,), i, jnp.int32)])
    return jnp.sum(jnp.where(iota16 == 0, g, 0))


def _pool_kernel(arr_hbm, gt_hbm, gs_out, hist_out,
                 idg_lin, ide_lin, vali_lin, cidx, scnt, bbuf0, bbuf1,
                 bbuf2, bbuf3, sums_v, hbuf, semg, seme, semg2, seme2, semh):
    wid = lax.axis_index("s") * NC + lax.axis_index("c")
    base = wid * FLATW
    iota16 = lax.iota(jnp.int32, LANES)

    pltpu.sync_copy(arr_hbm.at[pl.ds(base, FLATW)], idg_lin)
    pltpu.sync_copy(arr_hbm.at[pl.ds(B * L + base, FLATW)], ide_lin)
    pltpu.sync_copy(arr_hbm.at[pl.ds(2 * B * L + base, FLATW)], vali_lin)

    def compact_body(k, pos):
        sl = pl.ds(k * LANES, LANES)
        cidx[sl] = k * LANES + iota16
        m = vali_lin[sl] != 0
        mi = m.astype(jnp.int32)
        excl = plsc.cumsum(mi) - mi
        plsc.store_scatter(cidx, [excl + pos], idg_lin[sl], mask=m)
        return pos + jnp.sum(mi)
    nc = lax.fori_loop(0, FLATW // LANES, compact_body, jnp.int32(0))

    BUFS = (bbuf0, bbuf1, bbuf2, bbuf3)
    SEMS = (semg, seme, semg2, seme2)

    def fire(blk, b):
        buf, sem = BUFS[b], SEMS[b]
        @pl.when(blk * BLK < nc)
        def _():
            pltpu.async_copy(gt_hbm.at[cidx.at[pl.ds(blk * BLK, BLK)]],
                             buf, sem)

    def drain(blk, b):
        buf, sem = BUFS[b], SEMS[b]
        @pl.when(blk * BLK < nc)
        def _():
            pltpu.make_async_copy(gt_hbm.at[cidx.at[pl.ds(blk * BLK, BLK)]],
                                  buf, sem).wait()

    for _b in range(4):
        fire(_b, _b)

    rbase = jnp.int32(0)
    for bc in range(BPW // LANES):
        bvec50 = (iota16 + bc * LANES) * L
        def cnt_body(l, cnt):
            return cnt + plsc.load_gather(vali_lin, [bvec50 + l])
        cnt = lax.fori_loop(0, L, cnt_body, jnp.zeros((LANES,), jnp.int32))
        excl = plsc.cumsum(cnt) - cnt
        scnt[pl.ds(bc * LANES, LANES)] = excl + rbase
        rbase = rbase + jnp.sum(cnt)
    scnt[pl.ds(BPW, LANES)] = jnp.full((LANES,), rbase, jnp.int32)

    zero16 = jnp.zeros((LANES,), jnp.float32)
    lanebase = iota16 * D
    for bc in range(BPW // LANES):
        pb = bc % 2
        hrow = hbuf.at[pl.ds(pb * HW, HW)]
        if bc >= 2:
            for j in range(NB // D):
                hb = j * (B * D) + (wid * BPW + (bc - 2) * LANES) * D
                pltpu.make_async_copy(
                    hbuf.at[pl.ds(pb * HW + j * (LANES * D), LANES * D)],
                    hist_out.at[pl.ds(hb, LANES * D)], semh).wait()
        def zero_body(k, carry):
            for u in range(8):
                hrow[pl.ds((k * 8 + u) * LANES, LANES)] = zero16
            return carry
        lax.fori_loop(0, HW // (8 * LANES), zero_body, 0)
        bvec50 = (iota16 + bc * LANES) * L
        def fill_body(l, carry):
            bins = plsc.load_gather(ide_lin, [bvec50 + l])
            vals = plsc.load_gather(vali_lin, [bvec50 + l]).astype(jnp.float32)
            j = lax.shift_right_logical(bins, 7)
            rem = lax.bitwise_and(bins, jnp.int32(127))
            plsc.addupdate_scatter(hrow, [j * (LANES * D) + lanebase + rem],
                                   vals)
            return carry
        lax.fori_loop(0, L, fill_body, 0)
        for j in range(NB // D):
            hb = j * (B * D) + (wid * BPW + bc * LANES) * D
            pltpu.async_copy(
                hbuf.at[pl.ds(pb * HW + j * (LANES * D), LANES * D)],
                hist_out.at[pl.ds(hb, LANES * D)], semh)
    for bc in (BPW // LANES - 2, BPW // LANES - 1):
        for j in range(NB // D):
            hb = j * (B * D) + (wid * BPW + bc * LANES) * D
            pltpu.make_async_copy(
                hbuf.at[pl.ds((bc % 2) * HW + j * (LANES * D), LANES * D)],
                hist_out.at[pl.ds(hb, LANES * D)], semh).wait()

    def zs_body(r, carry):
        for dc in range(DC):
            sums_v[r, pl.ds(dc * LANES, LANES)] = zero16
        return carry
    lax.fori_loop(0, BPW, zs_body, 0)

    zeros8 = tuple(jnp.zeros((LANES,), jnp.float32) for _ in range(DC))

    def pair_body(fbp, st):
        for b in (0, 1, 2, 3):
            fb = fbp * 4 + b
            buf = BUFS[b]
            drain(fb, b)
            blockend = jnp.minimum((fb + 1) * BLK, nc)
            fb0 = fb * BLK

            def wcond(s):
                return s[1] < blockend

            def wbody(s):
                r, p = s
                e_r = _sread(scnt, r + 1, iota16)
                pe = jnp.minimum(e_r, blockend)
                off0 = p - fb0

                def abody(i, accs):
                    return tuple(
                        accs[dc] + buf[off0 + i, pl.ds(dc * LANES, LANES)]
                        for dc in range(DC))
                accs = lax.fori_loop(0, pe - p, abody, zeros8)
                for dc in range(DC):
                    sl = pl.ds(dc * LANES, LANES)
                    sums_v[r, sl] = sums_v[r, sl] + accs[dc]
                return (jnp.where(pe == e_r, r + 1, r), pe)

            st = lax.while_loop(wcond, wbody, st)
            fire(fb + 4, b)
        return st
    st = lax.fori_loop(0, NBLK // 4, pair_body,
                       (jnp.int32(0), jnp.int32(0)))
    for b in range(NBLK % 4):
        fb = (NBLK // 4) * 4 + b
        buf = BUFS[b]
        drain(fb, b)
        blockend = jnp.minimum((fb + 1) * BLK, nc)
        fb0 = fb * BLK

        def wcond2(s):
            return s[1] < blockend

        def wbody2(s):
            r, p = s
            e_r = _sread(scnt, r + 1, iota16)
            pe = jnp.minimum(e_r, blockend)
            off0 = p - fb0

            def abody2(i, accs):
                return tuple(
                    accs[dc] + buf[off0 + i, pl.ds(dc * LANES, LANES)]
                    for dc in range(DC))
            accs = lax.fori_loop(0, pe - p, abody2, zeros8)
            for dc in range(DC):
                sl = pl.ds(dc * LANES, LANES)
                sums_v[r, sl] = sums_v[r, sl] + accs[dc]
            return (jnp.where(pe == e_r, r + 1, r), pe)

        st = lax.while_loop(wcond2, wbody2, st)

    pltpu.sync_copy(sums_v, gs_out.at[pl.ds(wid * BPW, BPW)])


@jax.jit
def _sc_gather_hist(arr, gene_table):
    mesh = plsc.VectorSubcoreMesh(core_axis_name="c", subcore_axis_name="s",
                                  num_cores=NC, num_subcores=NS)
    return pl.kernel(
        _pool_kernel,
        out_type=(jax.ShapeDtypeStruct((B, D), jnp.float32),
                  jax.ShapeDtypeStruct((B * NB,), jnp.float32)),
        mesh=mesh,
        scratch_types=[
            pltpu.VMEM((FLATW,), jnp.int32),
            pltpu.VMEM((FLATW,), jnp.int32),
            pltpu.VMEM((FLATW,), jnp.int32),
            pltpu.VMEM((FLATW,), jnp.int32),
            pltpu.VMEM((BPW + LANES,), jnp.int32),
            pltpu.VMEM((BLK, D), jnp.float32),
            pltpu.VMEM((BLK, D), jnp.float32),
            pltpu.VMEM((BLK, D), jnp.float32),
            pltpu.VMEM((BLK, D), jnp.float32),
            pltpu.VMEM((BPW, D), jnp.float32),
            pltpu.VMEM((2 * HW,), jnp.float32),
            pltpu.SemaphoreType.DMA,
            pltpu.SemaphoreType.DMA,
            pltpu.SemaphoreType.DMA,
            pltpu.SemaphoreType.DMA,
            pltpu.SemaphoreType.DMA,
        ],
        compiler_params=pltpu.CompilerParams(needs_layout_passes=False),
    )(arr, gene_table)


RB = 512


def _combine_body(gs_ref, h_ref, et_ref, o_ref):
    h3 = h_ref[...]
    cnt = jnp.sum(h3, axis=(0, 2))[:, None]
    acc = gs_ref[...]
    for j in range(NB // D):
        acc = acc + lax.dot_general(
            h3[j], et_ref[pl.ds(j * D, D), :],
            (((1,), (0,)), ((), ())), preferred_element_type=jnp.float32)
    o_ref[...] = acc / jnp.maximum(cnt, 1.0)


@jax.jit
def _tc_combine(gs, histf, expr_table):
    hist3 = histf.reshape(NB // D, B, D)
    return pl.pallas_call(
        _combine_body,
        out_shape=jax.ShapeDtypeStruct((B, D), jnp.float32),
        grid=(B // RB,),
        in_specs=[
            pl.BlockSpec((RB, D), lambda i: (i, 0)),
            pl.BlockSpec((NB // D, RB, D), lambda i: (0, i, 0)),
            pl.BlockSpec((NB, D), lambda i: (0, 0)),
        ],
        out_specs=pl.BlockSpec((RB, D), lambda i: (i, 0)),
    )(gs, hist3, expr_table)


def kernel(identity_inputs, expression_inputs, attention_mask, gene_table,
           expr_table):
    arr = jnp.concatenate([
        identity_inputs.astype(jnp.int32).reshape(-1),
        expression_inputs.astype(jnp.int32).reshape(-1),
        (~attention_mask).astype(jnp.int32).reshape(-1),
    ])
    gs, histf = _sc_gather_hist(arr, gene_table.astype(jnp.float32))
    return _tc_combine(gs, histf, expr_table.astype(jnp.float32))
